# baseline (speedup 1.0000x reference)
import functools

import jax
import jax.numpy as jnp
from jax import lax
from jax.experimental import pallas as pl
from jax.experimental.pallas import tpu as pltpu
from jax.experimental.pallas import tpu_sc as plsc

NC = 2
NS = 16
NW = NC * NS
CHUNK = 128
DEGW = 128
BLK = 2048
NBUF = 2
NGRP = 2
R0 = 0.5


def _mm(a, w):
    return lax.dot_general(a, w, (((1,), (0,)), ((), ())),
                           preferred_element_type=jnp.float32)


def _tc1_body(x_ref, wp_ref, bp_ref, wl_ref, bl_ref, wr_ref,
              xp_ref, y1_ref, r1_ref):
    xb = x_ref[...]
    xp_ref[...] = jnp.maximum(_mm(xb, wp_ref[...]) + bp_ref[...], 0.0)
    y1_ref[...] = _mm(xb, wl_ref[...])
    r1_ref[...] = _mm(xb, wr_ref[...]) + bl_ref[...]


def _tc2_body(m_ref, deg_ref, xp_ref, r1_ref, wl_ref, bl_ref, wr_ref,
              h1_ref, y2_ref, r2_ref):
    rdeg = 1.0 / jnp.clip(deg_ref[0, :, 0:1] + deg_ref[1, :, 0:1], 1.0, None)
    h1 = jnp.maximum((m_ref[0] + m_ref[1]) * rdeg + r1_ref[...], 0.0)
    h1_ref[...] = h1
    xp = xp_ref[...]
    wl = wl_ref[...]
    wr = wr_ref[...]
    y2_ref[...] = _mm(xp, wl[0:128]) + _mm(h1, wl[128:256])
    r2_ref[...] = _mm(xp, wr[0:128]) + _mm(h1, wr[128:256]) + bl_ref[...]


def _tc3_body(m_ref, deg_ref, xp_ref, h1_ref, r2_ref, wl_ref, bl_ref, wr_ref,
              y3_ref, r3_ref):
    rdeg = 1.0 / jnp.clip(deg_ref[0, :, 0:1] + deg_ref[1, :, 0:1], 1.0, None)
    h2 = jnp.maximum((m_ref[0] + m_ref[1]) * rdeg + r2_ref[...], 0.0)
    xp = xp_ref[...]
    h1 = h1_ref[...]
    wl = wl_ref[...]
    wr = wr_ref[...]
    y3_ref[...] = (_mm(xp, wl[0:128]) + _mm(h1, wl[128:256])
                   + _mm(h2, wl[256:384]))
    r3_ref[...] = (_mm(xp, wr[0:128]) + _mm(h1, wr[128:256])
                   + _mm(h2, wr[256:384]) + bl_ref[...])


def _tc4_body(m_ref, deg_ref, r3_ref, h3_ref):
    rdeg = 1.0 / jnp.clip(deg_ref[0, :, 0:1] + deg_ref[1, :, 0:1], 1.0, None)
    h3_ref[...] = jnp.maximum((m_ref[0] + m_ref[1]) * rdeg + r3_ref[...], 0.0)


def _row_spec():
    return pl.BlockSpec((BLK, 128), lambda i: (i, 0))


def _full_spec(shape):
    nd = len(shape)
    return pl.BlockSpec(shape, lambda i: (0,) * nd)


def _part_spec(w):
    return pl.BlockSpec((2, BLK, w), lambda i: (0, i, 0))


def _tc_call(body, n_pad, in_specs, num_out):
    grid = n_pad // BLK
    out_shape = [jax.ShapeDtypeStruct((n_pad, 128), jnp.float32)] * num_out
    out_specs = [_row_spec()] * num_out
    return pl.pallas_call(
        body,
        grid=(grid,),
        in_specs=in_specs,
        out_specs=out_specs,
        out_shape=out_shape,
    )


def _mesh():
    return plsc.VectorSubcoreMesh(core_axis_name="c", subcore_axis_name="s",
                                  num_cores=NC, num_subcores=NS)


def _fill_rows(ref, nrows, ncols, value):
    @pl.loop(0, nrows)
    def _(r):
        for c in range(ncols // 16):
            ref[r, pl.ds(c * 16, 16)] = jnp.full((16,), value, jnp.float32)


def _seg_body(j0, jt, n_pad,
              y_hbm, src_hbm, dst_hbm, m_hbm,
              src_v, dst_v, rows, gsems, ssems, acc):
    cid = lax.axis_index("c")
    sid = lax.axis_index("s")

    rows_per_tile = n_pad // NS
    sl = pl.ds(sid * rows_per_tile, rows_per_tile)
    _fill_rows(rows[0], CHUNK, 128, 0.0)

    @pl.loop(0, rows_per_tile // CHUNK)
    def _(k):
        pltpu.sync_copy(rows[0],
                        acc.at[pl.ds(sid * rows_per_tile + k * CHUNK, CHUNK)])

    def gstart(j, b):
        pltpu.async_copy(y_hbm.at[src_v.at[j]], rows[b], gsems[b])

    def gwait(j, b):
        pltpu.make_async_copy(y_hbm.at[src_v.at[j]], rows[b], gsems[b]).wait()

    def sstart(j, b):
        pltpu.async_copy(rows[b], acc.at[dst_v.at[j]], ssems[b], add=True)

    def swait(j, b):
        pltpu.make_async_copy(rows[b], acc.at[dst_v.at[j]], ssems[b]).wait()

    plsc.subcore_barrier()

    def stage_idx(off, g, ib):
        pltpu.sync_copy(src_hbm.at[sid, pl.ds(off + g * ib, ib)],
                        src_v.at[pl.ds(0, ib)])
        pltpu.sync_copy(dst_hbm.at[sid, pl.ds(off + g * ib, ib)],
                        dst_v.at[pl.ds(0, ib)])

    def run_ring(jc, off):
        ib = jc // NGRP

        @pl.loop(0, NGRP)
        def _(g):
            stage_idx(off, g, ib)
            for b in range(NBUF):
                gstart(b, b)

            @pl.loop(0, ib // NBUF - 1)
            def _(k):
                j0 = k * NBUF
                for b in range(NBUF):
                    gwait(j0 + b, b)
                    sstart(j0 + b, b)
                for b in range(NBUF):
                    swait(j0 + b, b)
                    gstart(j0 + b + NBUF, b)

            jt = ib - NBUF
            for b in range(NBUF):
                gwait(jt + b, b)
                sstart(jt + b, b)
            for b in range(NBUF):
                swait(jt + b, b)

    def run_sync(jc, off):
        ib = jc // NGRP

        @pl.loop(0, NGRP)
        def _(g):
            stage_idx(off, g, ib)

            @pl.loop(0, ib)
            def _(j):
                pltpu.sync_copy(y_hbm.at[src_v.at[j]], rows[0])
                pltpu.sync_copy(rows[0], acc.at[dst_v.at[j]], add=True)

    @pl.when(cid == 0)
    def _():
        run_sync(j0, 0)

    @pl.when(cid == 1)
    def _():
        run_sync(jt - j0, j0)

    plsc.subcore_barrier()

    pltpu.sync_copy(acc.at[sl], m_hbm.at[cid, sl])


def _make_seg(j0, jt, n_pad):
    ibmax = max(j0, jt - j0) // NGRP
    scratch = [
        pltpu.VMEM((ibmax, CHUNK), jnp.int32),
        pltpu.VMEM((ibmax, CHUNK), jnp.int32),
        [pltpu.VMEM((CHUNK, 128), jnp.float32) for _ in range(NBUF)],
        [pltpu.SemaphoreType.DMA for _ in range(NBUF)],
        [pltpu.SemaphoreType.DMA for _ in range(NBUF)],
        pltpu.VMEM_SHARED((n_pad, 128), jnp.float32),
    ]
    body = functools.partial(_seg_body, j0, jt, n_pad)
    return pl.kernel(body,
                     out_type=jax.ShapeDtypeStruct((NC, n_pad, 128),
                                                   jnp.float32),
                     mesh=_mesh(), scratch_types=tuple(scratch))


def _deg_body(jt, n_pad,
              dst_hbm, deg_hbm,
              dst_v, ones_v, dsem, dega):
    cid = lax.axis_index("c")
    sid = lax.axis_index("s")
    jd = jt // NC

    pltpu.sync_copy(dst_hbm.at[sid, pl.ds(cid * jd, jd)], dst_v)

    rows_per_tile = n_pad // NS
    sl = pl.ds(sid * rows_per_tile, rows_per_tile)
    _fill_rows(ones_v, CHUNK, DEGW, 0.0)

    @pl.loop(0, rows_per_tile // CHUNK)
    def _(k):
        pltpu.sync_copy(ones_v,
                        dega.at[pl.ds(sid * rows_per_tile + k * CHUNK, CHUNK)])

    _fill_rows(ones_v, CHUNK, DEGW, 1.0)
    plsc.subcore_barrier()

    @pl.loop(0, jd // NBUF)
    def _(k):
        for b in range(NBUF):
            pltpu.async_copy(ones_v, dega.at[dst_v.at[k * NBUF + b]], dsem,
                             add=True)
        for b in range(NBUF):
            pltpu.make_async_copy(ones_v, dega.at[dst_v.at[k * NBUF + b]],
                                  dsem).wait()

    plsc.subcore_barrier()
    pltpu.sync_copy(dega.at[sl], deg_hbm.at[cid, sl])


def _make_deg(jt, n_pad):
    scratch = [
        pltpu.VMEM((jt // NC, CHUNK), jnp.int32),
        pltpu.VMEM((CHUNK, DEGW), jnp.float32),
        pltpu.SemaphoreType.DMA,
        pltpu.VMEM_SHARED((n_pad, DEGW), jnp.float32),
    ]
    body = functools.partial(_deg_body, jt, n_pad)
    return pl.kernel(body,
                     out_type=jax.ShapeDtypeStruct((NC, n_pad, DEGW),
                                                   jnp.float32),
                     mesh=_mesh(), scratch_types=tuple(scratch))


def kernel(x, edge_index, Wp, bp, Wl1, bl1, Wr1, Wl2, bl2, Wr2, Wl3, bl3, Wr3):
    n, d = x.shape
    e = edge_index.shape[1]
    n_pad = ((n + BLK - 1) // BLK) * BLK
    per = NS * CHUNK
    quant = NGRP * 8
    jt = (e + per - 1) // per
    jt = ((jt + quant - 1) // quant) * quant
    e_pad = jt * per
    j0 = int(round(jt * R0 / quant)) * quant
    j0 = min(max(j0, quant), jt - quant)

    src = edge_index[0]
    dst = edge_index[1]
    if e_pad != e:
        src = jnp.concatenate([src, jnp.zeros((e_pad - e,), jnp.int32)])
        dst = jnp.concatenate([dst, jnp.full((e_pad - e,), n, jnp.int32)])
    src_r = src.reshape(NS, jt, CHUNK)
    dst_r = dst.reshape(NS, jt, CHUNK)

    xpad = jnp.pad(x, ((0, n_pad - n), (0, 0)))
    bp2 = bp.reshape(1, 128)
    bl12 = bl1.reshape(1, 128)
    bl22 = bl2.reshape(1, 128)
    bl32 = bl3.reshape(1, 128)

    w128 = _full_spec((128, 128))
    w256 = _full_spec((256, 128))
    w384 = _full_spec((384, 128))
    b128 = _full_spec((1, 128))
    rows = _row_spec()
    mpart = _part_spec(128)
    dpart = _part_spec(DEGW)

    tc1 = _tc_call(_tc1_body, n_pad, [rows, w128, b128, w128, b128, w128], 3)
    xp, y1, r1 = tc1(xpad, Wp, bp2, Wl1, bl12, Wr1)

    deg = _make_deg(jt, n_pad)
    degp = deg(dst_r)

    seg = _make_seg(j0, jt, n_pad)
    m1p = seg(y1, src_r, dst_r)

    tc2 = _tc_call(_tc2_body, n_pad,
                   [mpart, dpart, rows, rows, w256, b128, w256], 3)
    h1, y2, r2 = tc2(m1p, degp, xp, r1, Wl2, bl22, Wr2)

    m2p = seg(y2, src_r, dst_r)

    tc3 = _tc_call(_tc3_body, n_pad,
                   [mpart, dpart, rows, rows, rows, w384, b128, w384], 2)
    y3, r3 = tc3(m2p, degp, xp, h1, r2, Wl3, bl32, Wr3)

    m3p = seg(y3, src_r, dst_r)

    tc4 = _tc_call(_tc4_body, n_pad, [mpart, dpart, rows], 1)
    (h3,) = tc4(m3p, degp, r3)

    return h3[:n]


# speedup vs baseline: 1.5555x; 1.5555x over previous
import functools

import jax
import jax.numpy as jnp
from jax import lax
from jax.experimental import pallas as pl
from jax.experimental.pallas import tpu as pltpu
from jax.experimental.pallas import tpu_sc as plsc

NC = 2
NS = 16
NW = NC * NS
CHUNK = 128
DEGW = 128
BLK = 2048


def _mm(a, w):
    return lax.dot_general(a, w, (((1,), (0,)), ((), ())),
                           preferred_element_type=jnp.float32)


def _tc1_body(x_ref, wp_ref, bp_ref, wl_ref, bl_ref, wr_ref,
              xp_ref, y1_ref, r1_ref):
    xb = x_ref[...]
    xp_ref[...] = jnp.maximum(_mm(xb, wp_ref[...]) + bp_ref[...], 0.0)
    y1_ref[...] = _mm(xb, wl_ref[...])
    r1_ref[...] = _mm(xb, wr_ref[...]) + bl_ref[...]


def _tc2_body(m_ref, deg_ref, xp_ref, r1_ref, wl_ref, bl_ref, wr_ref,
              h1_ref, y2_ref, r2_ref):
    rdeg = 1.0 / jnp.clip(deg_ref[0, :, 0:1] + deg_ref[1, :, 0:1], 1.0, None)
    h1 = jnp.maximum((m_ref[0] + m_ref[1]) * rdeg + r1_ref[...], 0.0)
    h1_ref[...] = h1
    xp = xp_ref[...]
    wl = wl_ref[...]
    wr = wr_ref[...]
    y2_ref[...] = _mm(xp, wl[0:128]) + _mm(h1, wl[128:256])
    r2_ref[...] = _mm(xp, wr[0:128]) + _mm(h1, wr[128:256]) + bl_ref[...]


def _tc3_body(m_ref, deg_ref, xp_ref, h1_ref, r2_ref, wl_ref, bl_ref, wr_ref,
              y3_ref, r3_ref):
    rdeg = 1.0 / jnp.clip(deg_ref[0, :, 0:1] + deg_ref[1, :, 0:1], 1.0, None)
    h2 = jnp.maximum((m_ref[0] + m_ref[1]) * rdeg + r2_ref[...], 0.0)
    xp = xp_ref[...]
    h1 = h1_ref[...]
    wl = wl_ref[...]
    wr = wr_ref[...]
    y3_ref[...] = (_mm(xp, wl[0:128]) + _mm(h1, wl[128:256])
                   + _mm(h2, wl[256:384]))
    r3_ref[...] = (_mm(xp, wr[0:128]) + _mm(h1, wr[128:256])
                   + _mm(h2, wr[256:384]) + bl_ref[...])


def _tc4_body(m_ref, deg_ref, r3_ref, h3_ref):
    rdeg = 1.0 / jnp.clip(deg_ref[0, :, 0:1] + deg_ref[1, :, 0:1], 1.0, None)
    h3_ref[...] = jnp.maximum((m_ref[0] + m_ref[1]) * rdeg + r3_ref[...], 0.0)


def _row_spec():
    return pl.BlockSpec((BLK, 128), lambda i: (i, 0))


def _full_spec(shape):
    nd = len(shape)
    return pl.BlockSpec(shape, lambda i: (0,) * nd)


def _part_spec(w):
    return pl.BlockSpec((2, BLK, w), lambda i: (0, i, 0))


def _tc_call(body, n_pad, in_specs, num_out):
    grid = n_pad // BLK
    out_shape = [jax.ShapeDtypeStruct((n_pad, 128), jnp.float32)] * num_out
    out_specs = [_row_spec()] * num_out
    return pl.pallas_call(
        body,
        grid=(grid,),
        in_specs=in_specs,
        out_specs=out_specs,
        out_shape=out_shape,
    )


def _mesh():
    return plsc.VectorSubcoreMesh(core_axis_name="c", subcore_axis_name="s",
                                  num_cores=NC, num_subcores=NS)


def _fill_rows(ref, nrows, ncols, value):
    @pl.loop(0, nrows)
    def _(r):
        for c in range(ncols // 16):
            ref[r, pl.ds(c * 16, 16)] = jnp.full((16,), value, jnp.float32)


def _seg_body(j_chunks, n_pad,
              y_hbm, src_hbm, dst_hbm, m_hbm,
              src_v, dst_v, rows_v, acc):
    cid = lax.axis_index("c")
    sid = lax.axis_index("s")
    wid = sid * NC + cid

    pltpu.sync_copy(src_hbm.at[wid], src_v)
    pltpu.sync_copy(dst_hbm.at[wid], dst_v)

    rows_per_tile = n_pad // NS
    sl = pl.ds(sid * rows_per_tile, rows_per_tile)
    _fill_rows(rows_v, CHUNK, 128, 0.0)

    @pl.loop(0, rows_per_tile // CHUNK)
    def _(k):
        pltpu.sync_copy(rows_v,
                        acc.at[pl.ds(sid * rows_per_tile + k * CHUNK, CHUNK)])

    plsc.subcore_barrier()

    @pl.loop(0, j_chunks)
    def _(j):
        pltpu.sync_copy(y_hbm.at[src_v.at[j]], rows_v)
        pltpu.sync_copy(rows_v, acc.at[dst_v.at[j]], add=True)

    plsc.subcore_barrier()

    pltpu.sync_copy(acc.at[sl], m_hbm.at[cid, sl])


def _make_seg(j_chunks, n_pad):
    scratch = [
        pltpu.VMEM((j_chunks, CHUNK), jnp.int32),
        pltpu.VMEM((j_chunks, CHUNK), jnp.int32),
        pltpu.VMEM((CHUNK, 128), jnp.float32),
        pltpu.VMEM_SHARED((n_pad, 128), jnp.float32),
    ]
    body = functools.partial(_seg_body, j_chunks, n_pad)
    return pl.kernel(body,
                     out_type=jax.ShapeDtypeStruct((NC, n_pad, 128),
                                                   jnp.float32),
                     mesh=_mesh(), scratch_types=tuple(scratch))


def _deg_body(j_chunks, n_pad,
              dst_hbm, deg_hbm,
              dst_v, ones_v, dega):
    cid = lax.axis_index("c")
    sid = lax.axis_index("s")
    wid = sid * NC + cid

    pltpu.sync_copy(dst_hbm.at[wid], dst_v)

    rows_per_tile = n_pad // NS
    sl = pl.ds(sid * rows_per_tile, rows_per_tile)
    _fill_rows(ones_v, CHUNK, DEGW, 0.0)

    @pl.loop(0, rows_per_tile // CHUNK)
    def _(k):
        pltpu.sync_copy(ones_v,
                        dega.at[pl.ds(sid * rows_per_tile + k * CHUNK, CHUNK)])

    _fill_rows(ones_v, CHUNK, DEGW, 1.0)
    plsc.subcore_barrier()

    @pl.loop(0, j_chunks)
    def _(j):
        pltpu.sync_copy(ones_v, dega.at[dst_v.at[j]], add=True)

    plsc.subcore_barrier()
    pltpu.sync_copy(dega.at[sl], deg_hbm.at[cid, sl])


def _make_deg(j_chunks, n_pad):
    scratch = [
        pltpu.VMEM((j_chunks, CHUNK), jnp.int32),
        pltpu.VMEM((CHUNK, DEGW), jnp.float32),
        pltpu.VMEM_SHARED((n_pad, DEGW), jnp.float32),
    ]
    body = functools.partial(_deg_body, j_chunks, n_pad)
    return pl.kernel(body,
                     out_type=jax.ShapeDtypeStruct((NC, n_pad, DEGW),
                                                   jnp.float32),
                     mesh=_mesh(), scratch_types=tuple(scratch))


def kernel(x, edge_index, Wp, bp, Wl1, bl1, Wr1, Wl2, bl2, Wr2, Wl3, bl3, Wr3):
    n, d = x.shape
    e = edge_index.shape[1]
    n_pad = ((n + BLK - 1) // BLK) * BLK
    per = NW * CHUNK
    j_chunks = (e + per - 1) // per
    e_pad = j_chunks * per

    src = edge_index[0]
    dst = edge_index[1]
    if e_pad != e:
        src = jnp.concatenate([src, jnp.zeros((e_pad - e,), jnp.int32)])
        dst = jnp.concatenate([dst, jnp.full((e_pad - e,), n, jnp.int32)])
    src_r = src.reshape(NW, j_chunks, CHUNK)
    dst_r = dst.reshape(NW, j_chunks, CHUNK)

    xpad = jnp.pad(x, ((0, n_pad - n), (0, 0)))
    bp2 = bp.reshape(1, 128)
    bl12 = bl1.reshape(1, 128)
    bl22 = bl2.reshape(1, 128)
    bl32 = bl3.reshape(1, 128)

    w128 = _full_spec((128, 128))
    w256 = _full_spec((256, 128))
    w384 = _full_spec((384, 128))
    b128 = _full_spec((1, 128))
    rows = _row_spec()
    mpart = _part_spec(128)
    dpart = _part_spec(DEGW)

    tc1 = _tc_call(_tc1_body, n_pad, [rows, w128, b128, w128, b128, w128], 3)
    xp, y1, r1 = tc1(xpad, Wp, bp2, Wl1, bl12, Wr1)

    deg = _make_deg(j_chunks, n_pad)
    degp = deg(dst_r)

    seg = _make_seg(j_chunks, n_pad)
    m1p = seg(y1, src_r, dst_r)

    tc2 = _tc_call(_tc2_body, n_pad,
                   [mpart, dpart, rows, rows, w256, b128, w256], 3)
    h1, y2, r2 = tc2(m1p, degp, xp, r1, Wl2, bl22, Wr2)

    m2p = seg(y2, src_r, dst_r)

    tc3 = _tc_call(_tc3_body, n_pad,
                   [mpart, dpart, rows, rows, rows, w384, b128, w384], 2)
    y3, r3 = tc3(m2p, degp, xp, h1, r2, Wl3, bl32, Wr3)

    m3p = seg(y3, src_r, dst_r)

    tc4 = _tc_call(_tc4_body, n_pad, [mpart, dpart, rows], 1)
    (h3,) = tc4(m3p, degp, r3)

    return h3[:n]
